# baseline (speedup 1.0000x reference)
import jax
import jax.numpy as jnp
from jax import lax
from jax.experimental import pallas as pl
from jax.experimental.pallas import tpu as pltpu

_N = 8192
_BI = 512
_BJ = 4096
_NI = _N // _BI
_NJ = _N // _BJ
_NSTEP = _NI * _NJ


def _norm_hi_lo(n):
    bf = jnp.bfloat16
    nh = n.astype(bf)
    nl = (n - nh.astype(jnp.float32)).astype(bf)
    return nh, nl


def _chamfer_block_kernel(p_ref, t_ref, out_ref, acache_ref, bcache_ref,
                          dbufa_ref, dbufb_ref, rowacc_ref, colacc_ref,
                          sum_ref):
    s = pl.program_id(0)
    i = jnp.minimum(s // _NJ, _NI - 1)
    j = lax.rem(s, _NJ)
    bf = jnp.bfloat16
    f32 = jnp.float32

    @pl.when(s == 0)
    def _():
        sum_ref[0] = 0.0
        t = t_ref[...]
        tx, ty, tz = t[0:1, :], t[1:2, :], t[2:3, :]
        tn = tx * tx + ty * ty + tz * tz
        tnh, tnl = _norm_hi_lo(tn)
        ones_t = jnp.ones((1, _N), bf)
        zeros_t = jnp.zeros((1, _N), bf)
        bfull = jnp.concatenate(
            [tx.astype(bf), ty.astype(bf), tz.astype(bf),
             ones_t, ones_t, tnh, tnl, zeros_t], axis=0)
        for jj in range(_NJ):
            bcache_ref[jj] = bfull[:, jj * _BJ:(jj + 1) * _BJ]

    @pl.when(j == 0)
    def _():
        p = p_ref[pl.ds(i * _BI, _BI), :]
        px, py, pz = p[:, 0:1], p[:, 1:2], p[:, 2:3]
        pn = px * px + py * py + pz * pz
        pnh, pnl = _norm_hi_lo(pn)
        ones_p = jnp.ones((_BI, 1), bf)
        zeros_p = jnp.zeros((_BI, 1), bf)
        acache_ref[...] = jnp.concatenate(
            [(-2.0 * px.astype(bf).astype(f32)).astype(bf),
             (-2.0 * py.astype(bf).astype(f32)).astype(bf),
             (-2.0 * pz.astype(bf).astype(f32)).astype(bf),
             pnh, pnl, ones_p, ones_p, zeros_p], axis=1)

    def _dot(dref):
        dref[...] = jax.lax.dot_general(
            acache_ref[...], bcache_ref[j], (((1,), (0,)), ((), ())),
            preferred_element_type=f32)

    def _reduce(dref):
        d = dref[...]
        rowm = jnp.min(d, axis=1, keepdims=True)
        colm = jnp.min(d, axis=0, keepdims=True)
        sp = s - 1
        ip = sp // _NJ
        jp = lax.rem(sp, _NJ)

        @pl.when(jp == 0)
        def _():
            rowacc_ref[...] = rowm

        @pl.when(jp > 0)
        def _():
            rowacc_ref[...] = jnp.minimum(rowacc_ref[...], rowm)

        @pl.when(jp == _NJ - 1)
        def _():
            sum_ref[0] += jnp.sum(rowacc_ref[...])

        @pl.when(ip == 0)
        def _():
            colacc_ref[jp] = colm

        @pl.when(ip > 0)
        def _():
            colacc_ref[jp] = jnp.minimum(colacc_ref[jp], colm)

    @pl.when(lax.rem(s, 2) == 0)
    def _():
        _reduce(dbufb_ref)
        _dot(dbufa_ref)

    @pl.when(lax.rem(s, 2) == 1)
    def _():
        _reduce(dbufa_ref)
        _dot(dbufb_ref)

    @pl.when(s == _NSTEP)
    def _():
        colsum = sum_ref[0]
        for jj in range(_NJ):
            colsum += jnp.sum(colacc_ref[jj])
        out_ref[...] = jnp.full((1, 1), colsum / (2.0 * _N), jnp.float32)


def _chamfer(pred, target_t, interpret=False):
    return pl.pallas_call(
        _chamfer_block_kernel,
        grid=(_NSTEP + 1,),
        in_specs=[
            pl.BlockSpec((_N, 3), lambda s: (0, 0)),
            pl.BlockSpec((3, _N), lambda s: (0, 0)),
        ],
        out_specs=pl.BlockSpec((1, 1), lambda s: (0, 0)),
        out_shape=jax.ShapeDtypeStruct((1, 1), jnp.float32),
        scratch_shapes=[
            pltpu.VMEM((_BI, 8), jnp.bfloat16),
            pltpu.VMEM((_NJ, 8, _BJ), jnp.bfloat16),
            pltpu.VMEM((_BI, _BJ), jnp.float32),
            pltpu.VMEM((_BI, _BJ), jnp.float32),
            pltpu.VMEM((_BI, 1), jnp.float32),
            pltpu.VMEM((_NJ, 1, _BJ), jnp.float32),
            pltpu.SMEM((1,), jnp.float32),
        ],
        interpret=interpret,
    )(pred, target_t)


@jax.jit
def kernel(pred_positions, target_positions):
    out = _chamfer(pred_positions, target_positions.T)
    return out[0, 0]


# speedup vs baseline: 1.0064x; 1.0064x over previous
import jax
import jax.numpy as jnp
from jax import lax
from jax.experimental import pallas as pl
from jax.experimental.pallas import tpu as pltpu

_N = 8192
_BI = 512
_BJ = 2048
_NI = _N // _BI
_NJ = _N // _BJ
_NSTEP = _NI * _NJ


def _norm_hi_lo(n):
    bf = jnp.bfloat16
    nh = n.astype(bf)
    nl = (n - nh.astype(jnp.float32)).astype(bf)
    return nh, nl


def _chamfer_block_kernel(p_ref, t_ref, out_ref, acache_ref, bcache_ref,
                          dbufa_ref, dbufb_ref, rowacc_ref, colacc_ref,
                          sum_ref):
    s = pl.program_id(0)
    i = jnp.minimum(s // _NJ, _NI - 1)
    j = lax.rem(s, _NJ)
    bf = jnp.bfloat16
    f32 = jnp.float32

    @pl.when(s == 0)
    def _():
        sum_ref[0] = 0.0
        t = t_ref[...]
        tx, ty, tz = t[0:1, :], t[1:2, :], t[2:3, :]
        tn = tx * tx + ty * ty + tz * tz
        tnh, tnl = _norm_hi_lo(tn)
        ones_t = jnp.ones((1, _N), bf)
        zeros_t = jnp.zeros((1, _N), bf)
        bfull = jnp.concatenate(
            [tx.astype(bf), ty.astype(bf), tz.astype(bf),
             ones_t, ones_t, tnh, tnl, zeros_t], axis=0)
        for jj in range(_NJ):
            bcache_ref[jj] = bfull[:, jj * _BJ:(jj + 1) * _BJ]

    @pl.when(j == 0)
    def _():
        p = p_ref[pl.ds(i * _BI, _BI), :]
        px, py, pz = p[:, 0:1], p[:, 1:2], p[:, 2:3]
        pn = px * px + py * py + pz * pz
        pnh, pnl = _norm_hi_lo(pn)
        ones_p = jnp.ones((_BI, 1), bf)
        zeros_p = jnp.zeros((_BI, 1), bf)
        acache_ref[...] = jnp.concatenate(
            [(-2.0 * px.astype(bf).astype(f32)).astype(bf),
             (-2.0 * py.astype(bf).astype(f32)).astype(bf),
             (-2.0 * pz.astype(bf).astype(f32)).astype(bf),
             pnh, pnl, ones_p, ones_p, zeros_p], axis=1)

    def _dot(dref):
        dref[...] = jax.lax.dot_general(
            acache_ref[...], bcache_ref[j], (((1,), (0,)), ((), ())),
            preferred_element_type=f32)

    def _reduce(dref):
        d = dref[...]
        rowm = jnp.min(d, axis=1, keepdims=True)
        colm = jnp.min(d, axis=0, keepdims=True)
        sp = s - 1
        ip = sp // _NJ
        jp = lax.rem(sp, _NJ)

        @pl.when(jp == 0)
        def _():
            rowacc_ref[...] = rowm

        @pl.when(jp > 0)
        def _():
            rowacc_ref[...] = jnp.minimum(rowacc_ref[...], rowm)

        @pl.when(jp == _NJ - 1)
        def _():
            sum_ref[0] += jnp.sum(rowacc_ref[...])

        @pl.when(ip == 0)
        def _():
            colacc_ref[jp] = colm

        @pl.when(ip > 0)
        def _():
            colacc_ref[jp] = jnp.minimum(colacc_ref[jp], colm)

    @pl.when(lax.rem(s, 2) == 0)
    def _():
        _reduce(dbufb_ref)
        _dot(dbufa_ref)

    @pl.when(lax.rem(s, 2) == 1)
    def _():
        _reduce(dbufa_ref)
        _dot(dbufb_ref)

    @pl.when(s == _NSTEP)
    def _():
        colsum = sum_ref[0]
        for jj in range(_NJ):
            colsum += jnp.sum(colacc_ref[jj])
        out_ref[...] = jnp.full((1, 1), colsum / (2.0 * _N), jnp.float32)


def _chamfer(pred, target_t, interpret=False):
    return pl.pallas_call(
        _chamfer_block_kernel,
        grid=(_NSTEP + 1,),
        in_specs=[
            pl.BlockSpec((_N, 3), lambda s: (0, 0)),
            pl.BlockSpec((3, _N), lambda s: (0, 0)),
        ],
        out_specs=pl.BlockSpec((1, 1), lambda s: (0, 0)),
        out_shape=jax.ShapeDtypeStruct((1, 1), jnp.float32),
        scratch_shapes=[
            pltpu.VMEM((_BI, 8), jnp.bfloat16),
            pltpu.VMEM((_NJ, 8, _BJ), jnp.bfloat16),
            pltpu.VMEM((_BI, _BJ), jnp.float32),
            pltpu.VMEM((_BI, _BJ), jnp.float32),
            pltpu.VMEM((_BI, 1), jnp.float32),
            pltpu.VMEM((_NJ, 1, _BJ), jnp.float32),
            pltpu.SMEM((1,), jnp.float32),
        ],
        interpret=interpret,
    )(pred, target_t)


@jax.jit
def kernel(pred_positions, target_positions):
    out = _chamfer(pred_positions, target_positions.T)
    return out[0, 0]


# speedup vs baseline: 1.1338x; 1.1266x over previous
import jax
import jax.numpy as jnp
from jax import lax
from jax.experimental import pallas as pl
from jax.experimental.pallas import tpu as pltpu

_N = 8192
_BI = 512
_BJ = 2048
_NI = _N // _BI
_NJ = _N // _BJ
_NSTEP = _NI * _NJ


def _norm_hi_lo(n):
    bf = jnp.bfloat16
    nh = n.astype(bf)
    nl = (n - nh.astype(jnp.float32)).astype(bf)
    return nh, nl


def _chamfer_block_kernel(p_ref, t_ref, out_ref, acache_ref, bcache_ref,
                          dbufa_ref, dbufb_ref, dbufc_ref, rowacc_ref,
                          colacc_ref, sum_ref):
    s = pl.program_id(0)
    i = jnp.minimum(s // _NJ, _NI - 1)
    j = lax.rem(s, _NJ)
    bf = jnp.bfloat16
    f32 = jnp.float32

    @pl.when(s == 0)
    def _():
        sum_ref[0] = 0.0
        t = t_ref[...]
        tx, ty, tz = t[0:1, :], t[1:2, :], t[2:3, :]
        tn = tx * tx + ty * ty + tz * tz
        tnh, tnl = _norm_hi_lo(tn)
        ones_t = jnp.ones((1, _N), bf)
        zeros_t = jnp.zeros((1, _N), bf)
        bfull = jnp.concatenate(
            [tx.astype(bf), ty.astype(bf), tz.astype(bf),
             ones_t, ones_t, tnh, tnl, zeros_t], axis=0)
        for jj in range(_NJ):
            bcache_ref[jj] = bfull[:, jj * _BJ:(jj + 1) * _BJ]

    @pl.when(j == 0)
    def _():
        p = p_ref[pl.ds(i * _BI, _BI), :]
        px, py, pz = p[:, 0:1], p[:, 1:2], p[:, 2:3]
        pn = px * px + py * py + pz * pz
        pnh, pnl = _norm_hi_lo(pn)
        ones_p = jnp.ones((_BI, 1), bf)
        zeros_p = jnp.zeros((_BI, 1), bf)
        acache_ref[...] = jnp.concatenate(
            [(-2.0 * px.astype(bf).astype(f32)).astype(bf),
             (-2.0 * py.astype(bf).astype(f32)).astype(bf),
             (-2.0 * pz.astype(bf).astype(f32)).astype(bf),
             pnh, pnl, ones_p, ones_p, zeros_p], axis=1)

    def _dot(dref):
        dref[...] = jax.lax.dot_general(
            acache_ref[...], bcache_ref[j], (((1,), (0,)), ((), ())),
            preferred_element_type=f32)

    def _reduce(dref):
        d = dref[...]
        rowm = jnp.min(d, axis=1, keepdims=True)
        colm = jnp.min(d, axis=0, keepdims=True)
        sp = s - 2
        ip = sp // _NJ
        jp = lax.rem(sp, _NJ)

        @pl.when(jp == 0)
        def _():
            rowacc_ref[...] = rowm

        @pl.when(jp > 0)
        def _():
            rowacc_ref[...] = jnp.minimum(rowacc_ref[...], rowm)

        @pl.when(jp == _NJ - 1)
        def _():
            sum_ref[0] += jnp.sum(rowacc_ref[...])

        @pl.when(ip == 0)
        def _():
            colacc_ref[jp] = colm

        @pl.when(ip > 0)
        def _():
            colacc_ref[jp] = jnp.minimum(colacc_ref[jp], colm)

    @pl.when(lax.rem(s, 3) == 0)
    def _():
        _dot(dbufa_ref)
        _reduce(dbufb_ref)

    @pl.when(lax.rem(s, 3) == 1)
    def _():
        _dot(dbufb_ref)
        _reduce(dbufc_ref)

    @pl.when(lax.rem(s, 3) == 2)
    def _():
        _dot(dbufc_ref)
        _reduce(dbufa_ref)

    @pl.when(s == _NSTEP + 1)
    def _():
        colsum = sum_ref[0]
        for jj in range(_NJ):
            colsum += jnp.sum(colacc_ref[jj])
        out_ref[...] = jnp.full((1, 1), colsum / (2.0 * _N), jnp.float32)


def _chamfer(pred, target_t, interpret=False):
    return pl.pallas_call(
        _chamfer_block_kernel,
        grid=(_NSTEP + 2,),
        in_specs=[
            pl.BlockSpec((_N, 3), lambda s: (0, 0)),
            pl.BlockSpec((3, _N), lambda s: (0, 0)),
        ],
        out_specs=pl.BlockSpec((1, 1), lambda s: (0, 0)),
        out_shape=jax.ShapeDtypeStruct((1, 1), jnp.float32),
        scratch_shapes=[
            pltpu.VMEM((_BI, 8), jnp.bfloat16),
            pltpu.VMEM((_NJ, 8, _BJ), jnp.bfloat16),
            pltpu.VMEM((_BI, _BJ), jnp.float32),
            pltpu.VMEM((_BI, _BJ), jnp.float32),
            pltpu.VMEM((_BI, _BJ), jnp.float32),
            pltpu.VMEM((_BI, 1), jnp.float32),
            pltpu.VMEM((_NJ, 1, _BJ), jnp.float32),
            pltpu.SMEM((1,), jnp.float32),
        ],
        interpret=interpret,
    )(pred, target_t)


@jax.jit
def kernel(pred_positions, target_positions):
    out = _chamfer(pred_positions, target_positions.T)
    return out[0, 0]


# speedup vs baseline: 1.1576x; 1.0210x over previous
import jax
import jax.numpy as jnp
from jax import lax
from jax.experimental import pallas as pl
from jax.experimental.pallas import tpu as pltpu

_N = 8192
_BI = 512
_BJ = 2048
_NI = _N // _BI
_NJ = _N // _BJ
_NSTEP = _NI * _NJ


def _norm_hi_lo(n):
    bf = jnp.bfloat16
    nh = n.astype(bf)
    nl = (n - nh.astype(jnp.float32)).astype(bf)
    return nh, nl


def _chamfer_block_kernel(p_ref, t_ref, out_ref, acache_ref, bcache_ref,
                          dbufa_ref, dbufb_ref, rowacc_ref, colacc_ref,
                          sum_ref):
    s = pl.program_id(0)
    i = jnp.minimum(s // _NJ, _NI - 1)
    j = lax.rem(s, _NJ)
    bf = jnp.bfloat16
    f32 = jnp.float32

    @pl.when(s == 0)
    def _():
        sum_ref[0] = 0.0
        t = t_ref[...]
        tx, ty, tz = t[0:1, :], t[1:2, :], t[2:3, :]
        tn = tx * tx + ty * ty + tz * tz
        tnh, tnl = _norm_hi_lo(tn)
        ones_t = jnp.ones((1, _N), bf)
        zeros_t = jnp.zeros((1, _N), bf)
        bfull = jnp.concatenate(
            [tx.astype(bf), ty.astype(bf), tz.astype(bf),
             ones_t, ones_t, tnh, tnl, zeros_t], axis=0)
        for jj in range(_NJ):
            bcache_ref[jj] = bfull[:, jj * _BJ:(jj + 1) * _BJ]

    @pl.when(j == 0)
    def _():
        p = p_ref[pl.ds(i * _BI, _BI), :]
        px, py, pz = p[:, 0:1], p[:, 1:2], p[:, 2:3]
        pn = px * px + py * py + pz * pz
        pnh, pnl = _norm_hi_lo(pn)
        ones_p = jnp.ones((_BI, 1), bf)
        zeros_p = jnp.zeros((_BI, 1), bf)
        acache_ref[...] = jnp.concatenate(
            [(-2.0 * px.astype(bf).astype(f32)).astype(bf),
             (-2.0 * py.astype(bf).astype(f32)).astype(bf),
             (-2.0 * pz.astype(bf).astype(f32)).astype(bf),
             pnh, pnl, ones_p, ones_p, zeros_p], axis=1)

    def _dot(dref):
        dref[...] = jax.lax.dot_general(
            acache_ref[...], bcache_ref[j], (((1,), (0,)), ((), ())),
            preferred_element_type=f32)

    def _reduce(dref):
        d = dref[...]
        rowm = jnp.min(d, axis=1, keepdims=True)
        colm = jnp.min(d.reshape(_BI // 8, 8, _BJ), axis=0)
        sp = s - 1
        ip = sp // _NJ
        jp = lax.rem(sp, _NJ)

        @pl.when(jp == 0)
        def _():
            rowacc_ref[...] = rowm

        @pl.when(jp > 0)
        def _():
            rowacc_ref[...] = jnp.minimum(rowacc_ref[...], rowm)

        @pl.when(jp == _NJ - 1)
        def _():
            sum_ref[0] += jnp.sum(rowacc_ref[...])

        @pl.when(ip == 0)
        def _():
            colacc_ref[jp] = colm

        @pl.when(ip > 0)
        def _():
            colacc_ref[jp] = jnp.minimum(colacc_ref[jp], colm)

    @pl.when(lax.rem(s, 2) == 0)
    def _():
        _dot(dbufa_ref)
        _reduce(dbufb_ref)

    @pl.when(lax.rem(s, 2) == 1)
    def _():
        _dot(dbufb_ref)
        _reduce(dbufa_ref)

    @pl.when(s == _NSTEP)
    def _():
        colsum = sum_ref[0]
        for jj in range(_NJ):
            colsum += jnp.sum(jnp.min(colacc_ref[jj], axis=0))
        out_ref[...] = jnp.full((1, 1), colsum / (2.0 * _N), jnp.float32)


def _chamfer(pred, target_t, interpret=False):
    return pl.pallas_call(
        _chamfer_block_kernel,
        grid=(_NSTEP + 1,),
        in_specs=[
            pl.BlockSpec((_N, 3), lambda s: (0, 0)),
            pl.BlockSpec((3, _N), lambda s: (0, 0)),
        ],
        out_specs=pl.BlockSpec((1, 1), lambda s: (0, 0)),
        out_shape=jax.ShapeDtypeStruct((1, 1), jnp.float32),
        scratch_shapes=[
            pltpu.VMEM((_BI, 8), jnp.bfloat16),
            pltpu.VMEM((_NJ, 8, _BJ), jnp.bfloat16),
            pltpu.VMEM((_BI, _BJ), jnp.float32),
            pltpu.VMEM((_BI, _BJ), jnp.float32),
            pltpu.VMEM((_BI, 1), jnp.float32),
            pltpu.VMEM((_NJ, 8, _BJ), jnp.float32),
            pltpu.SMEM((1,), jnp.float32),
        ],
        interpret=interpret,
    )(pred, target_t)


@jax.jit
def kernel(pred_positions, target_positions):
    out = _chamfer(pred_positions, target_positions.T)
    return out[0, 0]


# speedup vs baseline: 1.2640x; 1.0919x over previous
import jax
import jax.numpy as jnp
from jax import lax
from jax.experimental import pallas as pl
from jax.experimental.pallas import tpu as pltpu

_N = 8192
_BI = 1024
_BJ = 2048
_NI = _N // _BI
_NJ = _N // _BJ
_NSTEP = _NI * _NJ


def _norm_hi_lo(n):
    bf = jnp.bfloat16
    nh = n.astype(bf)
    nl = (n - nh.astype(jnp.float32)).astype(bf)
    return nh, nl


def _chamfer_block_kernel(p_ref, t_ref, out_ref, acache_ref, bcache_ref,
                          dbufa_ref, dbufb_ref, rowacc_ref, colacc_ref,
                          sum_ref):
    s = pl.program_id(0)
    i = jnp.minimum(s // _NJ, _NI - 1)
    j = lax.rem(s, _NJ)
    bf = jnp.bfloat16
    f32 = jnp.float32

    @pl.when(s == 0)
    def _():
        sum_ref[0] = 0.0
        t = t_ref[...]
        tx, ty, tz = t[0:1, :], t[1:2, :], t[2:3, :]
        tn = tx * tx + ty * ty + tz * tz
        tnh, tnl = _norm_hi_lo(tn)
        ones_t = jnp.ones((1, _N), bf)
        zeros_t = jnp.zeros((1, _N), bf)
        bfull = jnp.concatenate(
            [tx.astype(bf), ty.astype(bf), tz.astype(bf),
             ones_t, ones_t, tnh, tnl, zeros_t], axis=0)
        for jj in range(_NJ):
            bcache_ref[jj] = bfull[:, jj * _BJ:(jj + 1) * _BJ]

    @pl.when(j == 0)
    def _():
        p = p_ref[pl.ds(i * _BI, _BI), :]
        px, py, pz = p[:, 0:1], p[:, 1:2], p[:, 2:3]
        pn = px * px + py * py + pz * pz
        pnh, pnl = _norm_hi_lo(pn)
        ones_p = jnp.ones((_BI, 1), bf)
        zeros_p = jnp.zeros((_BI, 1), bf)
        acache_ref[...] = jnp.concatenate(
            [(-2.0 * px.astype(bf).astype(f32)).astype(bf),
             (-2.0 * py.astype(bf).astype(f32)).astype(bf),
             (-2.0 * pz.astype(bf).astype(f32)).astype(bf),
             pnh, pnl, ones_p, ones_p, zeros_p], axis=1)

    def _dot(dref):
        dref[...] = jax.lax.dot_general(
            acache_ref[...], bcache_ref[j], (((1,), (0,)), ((), ())),
            preferred_element_type=f32)

    def _reduce(dref):
        d = dref[...]
        rowm = jnp.min(d, axis=1, keepdims=True)
        colm = jnp.min(d.reshape(_BI // 8, 8, _BJ), axis=0)
        sp = s - 1
        ip = sp // _NJ
        jp = lax.rem(sp, _NJ)

        @pl.when(jp == 0)
        def _():
            rowacc_ref[...] = rowm

        @pl.when(jp > 0)
        def _():
            rowacc_ref[...] = jnp.minimum(rowacc_ref[...], rowm)

        @pl.when(jp == _NJ - 1)
        def _():
            sum_ref[0] += jnp.sum(rowacc_ref[...])

        @pl.when(ip == 0)
        def _():
            colacc_ref[jp] = colm

        @pl.when(ip > 0)
        def _():
            colacc_ref[jp] = jnp.minimum(colacc_ref[jp], colm)

    @pl.when(lax.rem(s, 2) == 0)
    def _():
        _dot(dbufa_ref)
        _reduce(dbufb_ref)

    @pl.when(lax.rem(s, 2) == 1)
    def _():
        _dot(dbufb_ref)
        _reduce(dbufa_ref)

    @pl.when(s == _NSTEP)
    def _():
        colsum = sum_ref[0]
        for jj in range(_NJ):
            colsum += jnp.sum(jnp.min(colacc_ref[jj], axis=0))
        out_ref[...] = jnp.full((1, 1), colsum / (2.0 * _N), jnp.float32)


def _chamfer(pred, target_t, interpret=False):
    return pl.pallas_call(
        _chamfer_block_kernel,
        grid=(_NSTEP + 1,),
        in_specs=[
            pl.BlockSpec((_N, 3), lambda s: (0, 0)),
            pl.BlockSpec((3, _N), lambda s: (0, 0)),
        ],
        out_specs=pl.BlockSpec((1, 1), lambda s: (0, 0)),
        out_shape=jax.ShapeDtypeStruct((1, 1), jnp.float32),
        scratch_shapes=[
            pltpu.VMEM((_BI, 8), jnp.bfloat16),
            pltpu.VMEM((_NJ, 8, _BJ), jnp.bfloat16),
            pltpu.VMEM((_BI, _BJ), jnp.float32),
            pltpu.VMEM((_BI, _BJ), jnp.float32),
            pltpu.VMEM((_BI, 1), jnp.float32),
            pltpu.VMEM((_NJ, 8, _BJ), jnp.float32),
            pltpu.SMEM((1,), jnp.float32),
        ],
        interpret=interpret,
    )(pred, target_t)


@jax.jit
def kernel(pred_positions, target_positions):
    out = _chamfer(pred_positions, target_positions.T)
    return out[0, 0]


# speedup vs baseline: 1.2803x; 1.0129x over previous
import jax
import jax.numpy as jnp
from jax import lax
from jax.experimental import pallas as pl
from jax.experimental.pallas import tpu as pltpu

_N = 8192
_BI = 2048
_BJ = 2048
_NI = _N // _BI
_NJ = _N // _BJ
_NSTEP = _NI * _NJ


def _norm_hi_lo(n):
    bf = jnp.bfloat16
    nh = n.astype(bf)
    nl = (n - nh.astype(jnp.float32)).astype(bf)
    return nh, nl


def _chamfer_block_kernel(p_ref, t_ref, out_ref, acache_ref, bcache_ref,
                          dbufa_ref, dbufb_ref, rowacc_ref, colacc_ref,
                          sum_ref):
    s = pl.program_id(0)
    i = jnp.minimum(s // _NJ, _NI - 1)
    j = lax.rem(s, _NJ)
    bf = jnp.bfloat16
    f32 = jnp.float32

    @pl.when(s == 0)
    def _():
        sum_ref[0] = 0.0
        t = t_ref[...]
        tx, ty, tz = t[0:1, :], t[1:2, :], t[2:3, :]
        tn = tx * tx + ty * ty + tz * tz
        tnh, tnl = _norm_hi_lo(tn)
        ones_t = jnp.ones((1, _N), bf)
        zeros_t = jnp.zeros((1, _N), bf)
        bfull = jnp.concatenate(
            [tx.astype(bf), ty.astype(bf), tz.astype(bf),
             ones_t, ones_t, tnh, tnl, zeros_t], axis=0)
        for jj in range(_NJ):
            bcache_ref[jj] = bfull[:, jj * _BJ:(jj + 1) * _BJ]

    @pl.when(j == 0)
    def _():
        p = p_ref[pl.ds(i * _BI, _BI), :]
        px, py, pz = p[:, 0:1], p[:, 1:2], p[:, 2:3]
        pn = px * px + py * py + pz * pz
        pnh, pnl = _norm_hi_lo(pn)
        ones_p = jnp.ones((_BI, 1), bf)
        zeros_p = jnp.zeros((_BI, 1), bf)
        acache_ref[...] = jnp.concatenate(
            [(-2.0 * px.astype(bf).astype(f32)).astype(bf),
             (-2.0 * py.astype(bf).astype(f32)).astype(bf),
             (-2.0 * pz.astype(bf).astype(f32)).astype(bf),
             pnh, pnl, ones_p, ones_p, zeros_p], axis=1)

    def _dot(dref):
        dref[...] = jax.lax.dot_general(
            acache_ref[...], bcache_ref[j], (((1,), (0,)), ((), ())),
            preferred_element_type=f32)

    def _reduce(dref):
        d = dref[...]
        rowm = jnp.min(d, axis=1, keepdims=True)
        colm = jnp.min(d.reshape(_BI // 8, 8, _BJ), axis=0)
        sp = s - 1
        ip = sp // _NJ
        jp = lax.rem(sp, _NJ)

        @pl.when(jp == 0)
        def _():
            rowacc_ref[...] = rowm

        @pl.when(jp > 0)
        def _():
            rowacc_ref[...] = jnp.minimum(rowacc_ref[...], rowm)

        @pl.when(jp == _NJ - 1)
        def _():
            sum_ref[0] += jnp.sum(rowacc_ref[...])

        @pl.when(ip == 0)
        def _():
            colacc_ref[jp] = colm

        @pl.when(ip > 0)
        def _():
            colacc_ref[jp] = jnp.minimum(colacc_ref[jp], colm)

    @pl.when(lax.rem(s, 2) == 0)
    def _():
        _dot(dbufa_ref)
        _reduce(dbufb_ref)

    @pl.when(lax.rem(s, 2) == 1)
    def _():
        _dot(dbufb_ref)
        _reduce(dbufa_ref)

    @pl.when(s == _NSTEP)
    def _():
        colsum = sum_ref[0]
        for jj in range(_NJ):
            colsum += jnp.sum(jnp.min(colacc_ref[jj], axis=0))
        out_ref[...] = jnp.full((1, 1), colsum / (2.0 * _N), jnp.float32)


def _chamfer(pred, target_t, interpret=False):
    return pl.pallas_call(
        _chamfer_block_kernel,
        grid=(_NSTEP + 1,),
        in_specs=[
            pl.BlockSpec((_N, 3), lambda s: (0, 0)),
            pl.BlockSpec((3, _N), lambda s: (0, 0)),
        ],
        out_specs=pl.BlockSpec((1, 1), lambda s: (0, 0)),
        out_shape=jax.ShapeDtypeStruct((1, 1), jnp.float32),
        scratch_shapes=[
            pltpu.VMEM((_BI, 8), jnp.bfloat16),
            pltpu.VMEM((_NJ, 8, _BJ), jnp.bfloat16),
            pltpu.VMEM((_BI, _BJ), jnp.float32),
            pltpu.VMEM((_BI, _BJ), jnp.float32),
            pltpu.VMEM((_BI, 1), jnp.float32),
            pltpu.VMEM((_NJ, 8, _BJ), jnp.float32),
            pltpu.SMEM((1,), jnp.float32),
        ],
        interpret=interpret,
    )(pred, target_t)


@jax.jit
def kernel(pred_positions, target_positions):
    out = _chamfer(pred_positions, target_positions.T)
    return out[0, 0]
